# Initial kernel scaffold; baseline (speedup 1.0000x reference)
import jax
import jax.numpy as jnp
from jax.experimental import pallas as pl


def kernel(inputs, word_table, pos_table):
    raise NotImplementedError("write your pallas kernel here")


# speedup vs baseline: 1.2990x; 1.2990x over previous
import functools

import jax
import jax.numpy as jnp
from jax import lax
from jax.experimental import pallas as pl
from jax.experimental.pallas import tpu as pltpu
from jax.experimental.pallas import tpu_sc as plsc

B = 4096
L = 200
D = 32
ROWS = B * L
NC = 2
NS = 16
NW = NC * NS
ROWS_PER_W = ROWS // NW
CHUNK = 1024
N_CHUNKS = ROWS_PER_W // CHUNK
G = 8
GI = CHUNK // G
POS_EXT = 1224

_mesh = plsc.VectorSubcoreMesh(core_axis_name="c", subcore_axis_name="s")


@functools.partial(
    pl.kernel,
    out_type=jax.ShapeDtypeStruct((ROWS, D), jnp.float32),
    mesh=_mesh,
    scratch_types=[
        pltpu.VMEM((G, GI), jnp.int32),
        pltpu.VMEM((CHUNK, D), jnp.float32),
        pltpu.VMEM((POS_EXT, D), jnp.float32),
        pltpu.SemaphoreType.DMA,
    ],
    compiler_params=pltpu.CompilerParams(use_tc_tiling_on_sc=False),
)
def _sc_embed(idx_hbm, table_hbm, posext_hbm, out_hbm, idxv, rowbuf, posv, gsem):
    wid = lax.axis_index("s") * NC + lax.axis_index("c")
    pltpu.sync_copy(posext_hbm, posv)
    row0 = wid * ROWS_PER_W
    irow0 = wid * (ROWS_PER_W // GI)
    for c in range(N_CHUNKS):
        base = row0 + c * CHUNK
        pltpu.sync_copy(idx_hbm.at[pl.ds(irow0 + c * G, G)], idxv)
        handles = [
            pltpu.async_copy(
                table_hbm.at[idxv.at[j]], rowbuf.at[pl.ds(j * GI, GI)], gsem
            )
            for j in range(G)
        ]
        for h in handles:
            h.wait()
        phi = (c * CHUNK) % L

        def add_body(r, _, phi=phi):
            lo = rowbuf[r, pl.ds(0, 16)] + posv[phi + r, pl.ds(0, 16)]
            rowbuf[r, pl.ds(0, 16)] = lo
            hi = rowbuf[r, pl.ds(16, 16)] + posv[phi + r, pl.ds(16, 16)]
            rowbuf[r, pl.ds(16, 16)] = hi
            return 0

        lax.fori_loop(0, CHUNK, add_body, 0)
        pltpu.sync_copy(rowbuf, out_hbm.at[pl.ds(base, CHUNK)])


def kernel(inputs, word_table, pos_table):
    idx = inputs.reshape(ROWS).astype(jnp.int32).reshape(ROWS // GI, GI)
    reps = -(-POS_EXT // L)
    posext = jnp.tile(pos_table, (reps, 1))[:POS_EXT]
    out = _sc_embed(idx, word_table, posext)
    return out.reshape(B, L, D)


# speedup vs baseline: 1.4313x; 1.1019x over previous
import functools

import jax
import jax.numpy as jnp
from jax import lax
from jax.experimental import pallas as pl
from jax.experimental.pallas import tpu as pltpu
from jax.experimental.pallas import tpu_sc as plsc

B = 4096
L = 200
D = 32
ROWS = B * L
NC = 2
NS = 16
NW = NC * NS
ROWS_PER_W = ROWS // NW
CHUNK = 1024
N_CHUNKS = ROWS_PER_W // CHUNK
G = 8
GI = CHUNK // G
POS_EXT = 1224

_mesh = plsc.VectorSubcoreMesh(core_axis_name="c", subcore_axis_name="s")


@functools.partial(
    pl.kernel,
    out_type=jax.ShapeDtypeStruct((ROWS, D), jnp.float32),
    mesh=_mesh,
    scratch_types=[
        pltpu.VMEM((G, GI), jnp.int32),
        pltpu.VMEM((G, GI), jnp.int32),
        pltpu.VMEM((CHUNK, D), jnp.float32),
        pltpu.VMEM((CHUNK, D), jnp.float32),
        pltpu.VMEM((POS_EXT, D), jnp.float32),
        pltpu.SemaphoreType.DMA,
        pltpu.SemaphoreType.DMA,
        pltpu.SemaphoreType.DMA,
        pltpu.SemaphoreType.DMA,
    ],
    compiler_params=pltpu.CompilerParams(use_tc_tiling_on_sc=False),
)
def _sc_embed(idx_hbm, table_hbm, posext_hbm, out_hbm,
              idxv0, idxv1, rowbuf0, rowbuf1, posv,
              gsem0, gsem1, ssem0, ssem1):
    wid = lax.axis_index("s") * NC + lax.axis_index("c")
    pltpu.sync_copy(posext_hbm, posv)
    row0 = wid * ROWS_PER_W
    irow0 = wid * (ROWS_PER_W // GI)

    idxvs = [idxv0, idxv1]
    rowbufs = [rowbuf0, rowbuf1]
    gsems = [gsem0, gsem1]
    ssems = [ssem0, ssem1]

    def load_and_fire(c):
        p = c % 2
        pltpu.sync_copy(idx_hbm.at[pl.ds(irow0 + c * G, G)], idxvs[p])
        return [
            pltpu.async_copy(
                table_hbm.at[idxvs[p].at[j]],
                rowbufs[p].at[pl.ds(j * GI, GI)],
                gsems[p],
            )
            for j in range(G)
        ]

    gather_handles = {0: load_and_fire(0)}
    store_handles = {}
    for c in range(N_CHUNKS):
        p = c % 2
        if c + 1 < N_CHUNKS:
            if c >= 1:
                store_handles.pop(c - 1).wait()
            gather_handles[c + 1] = load_and_fire(c + 1)
        for h in gather_handles.pop(c):
            h.wait()

        phi = (c * CHUNK) % L
        rb = rowbufs[p]

        @plsc.parallel_loop(0, CHUNK, unroll=8)
        def add_body(r, phi=phi, rb=rb):
            rb[r, pl.ds(0, 16)] += posv[phi + r, pl.ds(0, 16)]
            rb[r, pl.ds(16, 16)] += posv[phi + r, pl.ds(16, 16)]

        store_handles[c] = pltpu.async_copy(
            rb, out_hbm.at[pl.ds(row0 + c * CHUNK, CHUNK)], ssems[p]
        )
    for c in sorted(store_handles):
        store_handles[c].wait()


def kernel(inputs, word_table, pos_table):
    idx = inputs.reshape(ROWS).astype(jnp.int32).reshape(ROWS // GI, GI)
    reps = -(-POS_EXT // L)
    posext = jnp.tile(pos_table, (reps, 1))[:POS_EXT]
    out = _sc_embed(idx, word_table, posext)
    return out.reshape(B, L, D)
